# baseline (speedup 1.0000x reference)
import functools

import jax
import jax.numpy as jnp
from jax import lax
from jax.experimental import pallas as pl
from jax.experimental.pallas import tpu as pltpu
from jax.experimental.pallas import tpu_sc as plsc

N = 100000
NHID = 128
NC = 2
NS = 16
NW = NC * NS
C = 128
SROWS = 2 * C
FULL = N // C
CPW = FULL // NW
NSUP = CPW // 2
EXTRA = FULL - CPW * NW
TAIL_ROWS = N - FULL * C
TAILW = NW - 1
KMAX = CPW + 1

_mesh = plsc.VectorSubcoreMesh(core_axis_name="c", subcore_axis_name="s")


@functools.partial(
    pl.kernel,
    out_type=jax.ShapeDtypeStruct((N, NHID), jnp.float32),
    mesh=_mesh,
    scratch_types=[
        pltpu.VMEM((KMAX * C,), jnp.int32),
        pltpu.VMEM((SROWS, NHID), jnp.float32),
        pltpu.VMEM((SROWS, NHID), jnp.float32),
        pltpu.VMEM((C, NHID), jnp.float32),
        pltpu.VMEM((C, NHID), jnp.float32),
        pltpu.VMEM((C, NHID), jnp.float32),
        pltpu.SemaphoreType.DMA,
        pltpu.SemaphoreType.DMA,
        pltpu.SemaphoreType.DMA,
        pltpu.SemaphoreType.DMA,
        pltpu.SemaphoreType.DMA,
        pltpu.SemaphoreType.DMA,
        pltpu.SemaphoreType.DMA,
    ],
)
def _wl_pe(h_hbm, idx_hbm, w_hbm, out_hbm,
           idx_v, r0b, r1b, g0, g1, g2, sh0, sh1, sg0, sg1, sg2, t0, t1):
    wid = lax.axis_index("s") * NC + lax.axis_index("c")
    b0 = jnp.where(
        wid < EXTRA, KMAX * wid, KMAX * EXTRA + CPW * (wid - EXTRA)
    ).astype(jnp.int32)

    @pl.when(wid < EXTRA)
    def _load_idx_25():
        src = pl.ds(pl.multiple_of(b0 * C, C), KMAX * C)
        pltpu.sync_copy(idx_hbm.at[src], idx_v)

    @pl.when(wid >= EXTRA)
    def _load_idx_24():
        src = pl.ds(pl.multiple_of(b0 * C, C), CPW * C)
        pltpu.sync_copy(idx_hbm.at[src], idx_v.at[pl.ds(0, CPW * C)])

    @pl.when(wid == TAILW)
    def _load_idx_tail():
        src = pl.ds(pl.multiple_of(FULL * C, C), TAIL_ROWS)
        pltpu.sync_copy(idx_hbm.at[src], idx_v.at[pl.ds(CPW * C, TAIL_ROWS)])

    rb, gb = (r0b, r1b), (g0, g1, g2)
    shb, sgb, tb = (sh0, sh1), (sg0, sg1, sg2), (t0, t1)

    def chunk_row0(k):
        return pl.multiple_of((b0 + k) * C, C)

    def start_h(s, b):
        r0 = chunk_row0(2 * s)
        pltpu.async_copy(h_hbm.at[pl.ds(r0, SROWS)], rb[b], shb[b])

    def wait_h(b):
        pltpu.make_async_copy(h_hbm.at[pl.ds(0, SROWS)], rb[b], shb[b]).wait()

    def start_g(k, gs):
        idx_ref = idx_v.at[pl.ds(k * C, C)]
        pltpu.async_copy(w_hbm.at[idx_ref], gb[gs], sgb[gs])

    def wait_g(gs):
        pltpu.make_async_copy(h_hbm.at[pl.ds(0, C)], gb[gs], sgb[gs]).wait()

    def add_half(b, j, gs, rows=C):
        def add_row(r, carry):
            for jj in range(NHID // 16):
                sl = pl.ds(jj * 16, 16)
                plsc.addupdate(rb[b].at[j * C + r, sl], gb[gs][r, sl])
            return carry

        lax.fori_loop(0, rows, add_row, 0)

    def start_out(s, b):
        r0 = chunk_row0(2 * s)
        pltpu.async_copy(rb[b], out_hbm.at[pl.ds(r0, SROWS)], tb[b])

    def wait_t(b):
        pltpu.make_async_copy(h_hbm.at[pl.ds(0, SROWS)], rb[b], tb[b]).wait()

    start_h(0, 0)
    start_g(0, 0)
    start_g(1, 1)
    start_g(2, 2)

    for s in range(NSUP):
        b = s % 2
        k = 2 * s
        wait_h(b)
        for j in (0, 1):
            gs = (k + j) % 3
            wait_g(gs)
            add_half(b, j, gs)
            nxt = k + j + 3
            if nxt < CPW:
                start_g(nxt, gs)
            elif nxt == CPW:
                @pl.when(wid < EXTRA)
                def _start_extra_gather(nxt=nxt, gs=gs):
                    start_g(nxt, gs)
        start_out(s, b)
        if s >= 1:
            wait_t(1 - b)
        if s + 1 < NSUP:
            start_h(s + 1, 1 - b)
        if s + 1 == NSUP:
            @pl.when(wid < EXTRA)
            def _start_extra_h():
                r0 = chunk_row0(CPW)
                pltpu.async_copy(h_hbm.at[pl.ds(r0, C)],
                                 r0b.at[pl.ds(0, C)], sh0)

    @pl.when(wid < EXTRA)
    def _extra():
        pltpu.make_async_copy(h_hbm.at[pl.ds(0, C)],
                              r0b.at[pl.ds(0, C)], sh0).wait()
        wait_g(0)
        add_half(0, 0, 0)
        r0 = chunk_row0(CPW)
        pltpu.sync_copy(r0b.at[pl.ds(0, C)], out_hbm.at[pl.ds(r0, C)])

    @pl.when(wid == TAILW)
    def _tail():
        r0 = pl.multiple_of(FULL * C, C)
        t = pl.ds(0, TAIL_ROWS)
        pltpu.sync_copy(h_hbm.at[pl.ds(r0, TAIL_ROWS)], r0b.at[t])
        idx_ref = idx_v.at[pl.ds(CPW * C, TAIL_ROWS)]
        pltpu.async_copy(w_hbm.at[idx_ref], g0.at[t], sg0).wait()
        add_half(0, 0, 0, rows=TAIL_ROWS)
        pltpu.sync_copy(r0b.at[t], out_hbm.at[pl.ds(r0, TAIL_ROWS)])

    wait_t(1)


def kernel(h, precomputed_eigenvectors, proj_weight):
    idx = precomputed_eigenvectors.astype(jnp.int32)
    return _wl_pe(h, idx, proj_weight)


# speedup vs baseline: 1.2149x; 1.2149x over previous
import functools

import jax
import jax.numpy as jnp
from jax import lax
from jax.experimental import pallas as pl
from jax.experimental.pallas import tpu as pltpu
from jax.experimental.pallas import tpu_sc as plsc

N = 100000
NHID = 128
NC = 2
NS = 16
NW = NC * NS
C = 128
FULL = N // C
CPW = FULL // NW
EXTRA = FULL - CPW * NW
TAIL_ROWS = N - FULL * C
TAILW = NW - 1
KMAX = CPW + 1

_mesh = plsc.VectorSubcoreMesh(core_axis_name="c", subcore_axis_name="s")


@functools.partial(
    pl.kernel,
    out_type=jax.ShapeDtypeStruct((N, NHID), jnp.float32),
    mesh=_mesh,
    scratch_types=[
        pltpu.VMEM((KMAX * C,), jnp.int32),
        pltpu.VMEM((C, NHID), jnp.float32),
        pltpu.VMEM((C, NHID), jnp.float32),
        pltpu.VMEM((C, NHID), jnp.float32),
        pltpu.VMEM((C, NHID), jnp.float32),
        pltpu.VMEM((C, NHID), jnp.float32),
        pltpu.VMEM((C, NHID), jnp.float32),
        pltpu.SemaphoreType.DMA,
        pltpu.SemaphoreType.DMA,
        pltpu.SemaphoreType.DMA,
        pltpu.SemaphoreType.DMA,
        pltpu.SemaphoreType.DMA,
        pltpu.SemaphoreType.DMA,
    ],
)
def _wl_pe(h_hbm, idx_hbm, w_hbm, out_hbm,
           idx_v, r0b, r1b, r2b, g0, g1, g2, s0, s1, s2, t0, t1, t2):
    wid = lax.axis_index("s") * NC + lax.axis_index("c")
    b0 = jnp.where(
        wid < EXTRA, KMAX * wid, KMAX * EXTRA + CPW * (wid - EXTRA)
    ).astype(jnp.int32)

    @pl.when(wid < EXTRA)
    def _load_idx_25():
        src = pl.ds(pl.multiple_of(b0 * C, C), KMAX * C)
        pltpu.sync_copy(idx_hbm.at[src], idx_v)

    @pl.when(wid >= EXTRA)
    def _load_idx_24():
        src = pl.ds(pl.multiple_of(b0 * C, C), CPW * C)
        pltpu.sync_copy(idx_hbm.at[src], idx_v.at[pl.ds(0, CPW * C)])

    @pl.when(wid == TAILW)
    def _load_idx_tail():
        src = pl.ds(pl.multiple_of(FULL * C, C), TAIL_ROWS)
        pltpu.sync_copy(idx_hbm.at[src], idx_v.at[pl.ds(CPW * C, TAIL_ROWS)])

    rb, gb = (r0b, r1b, r2b), (g0, g1, g2)
    sb, tb = (s0, s1, s2), (t0, t1, t2)

    def row0_of(k):
        return pl.multiple_of((b0 + k) * C, C)

    def start_in(k, b):
        r0 = row0_of(k)
        pltpu.async_copy(h_hbm.at[pl.ds(r0, C)], rb[b], sb[b])
        idx_ref = idx_v.at[pl.ds(k * C, C)]
        pltpu.async_copy(w_hbm.at[idx_ref], gb[b], sb[b])

    def wait_in(b):
        pltpu.make_async_copy(h_hbm.at[pl.ds(0, C)], rb[b], sb[b]).wait()
        pltpu.make_async_copy(h_hbm.at[pl.ds(0, C)], gb[b], sb[b]).wait()

    def add(b, rows=C):
        def add_row(r, carry):
            for j in range(NHID // 16):
                sl = pl.ds(j * 16, 16)
                plsc.addupdate(rb[b].at[r, sl], gb[b][r, sl])
            return carry

        lax.fori_loop(0, rows, add_row, 0)

    def start_out(k, b):
        pltpu.async_copy(rb[b], out_hbm.at[pl.ds(row0_of(k), C)], tb[b])

    def wait_out(b):
        pltpu.make_async_copy(h_hbm.at[pl.ds(0, C)], rb[b], tb[b]).wait()

    start_in(0, 0)
    start_in(1, 1)
    wait_in(0)
    add(0)
    start_out(0, 0)
    start_in(2, 2)

    def step(k, b):
        wait_in(b)
        add(b)
        start_out(k, b)
        b2 = (b + 2) % 3
        wait_out(b2)
        start_in(k + 2, b2)

    def triple(i, carry):
        k = 3 * i + 1
        step(k, 1)
        step(k + 1, 2)
        step(k + 2, 0)
        return carry

    lax.fori_loop(0, (CPW - 3) // 3, triple, 0)

    wait_in(1)
    add(1)
    start_out(CPW - 2, 1)
    wait_out(0)

    @pl.when(wid < EXTRA)
    def _start_extra():
        start_in(CPW, 0)

    wait_in(2)
    add(2)
    start_out(CPW - 1, 2)

    @pl.when(wid < EXTRA)
    def _finish_extra():
        wait_in(0)
        add(0)
        start_out(CPW, 0)
        wait_out(0)

    @pl.when(wid == TAILW)
    def _tail():
        r0 = pl.multiple_of(FULL * C, C)
        t = pl.ds(0, TAIL_ROWS)
        pltpu.sync_copy(h_hbm.at[pl.ds(r0, TAIL_ROWS)], r0b.at[t])
        idx_ref = idx_v.at[pl.ds(CPW * C, TAIL_ROWS)]
        pltpu.async_copy(w_hbm.at[idx_ref], g0.at[t], s0).wait()
        add(0, rows=TAIL_ROWS)
        pltpu.sync_copy(r0b.at[t], out_hbm.at[pl.ds(r0, TAIL_ROWS)])

    wait_out(1)
    wait_out(2)


def kernel(h, precomputed_eigenvectors, proj_weight):
    idx = precomputed_eigenvectors.astype(jnp.int32)
    return _wl_pe(h, idx, proj_weight)
